# baseline (speedup 1.0000x reference)
import dataclasses
import functools

import jax
import jax.numpy as jnp
from jax import lax
from jax.experimental import pallas as pl
from jax.experimental.pallas import tpu as pltpu
from jax.experimental.pallas import tpu_sc as plsc

SEQLEN = 50
BATCH = 4096
D = 64
NTYPES_N = 100
NTYPES_E = 50
NROWS = SEQLEN * BATCH
TOTNNZ = 409600
NBLK = 20
RB = NROWS // NBLK
CHUNK = 256
NCHUNK = RB // CHUNK
WB = 512
NC = 2
NS = 16
ROWS_PER_TILE = RB // NS


def _prep_body(node_ref, edge_ref, posn_ref, pose_ref, posv_ref,
               pr_ref, tab_ref, pos_ref):
    pr_ref[...] = jnp.broadcast_to(posv_ref[...][:, None, :],
                                   (SEQLEN, CHUNK, D))
    node_t = node_ref[...].T
    edge_t = jnp.pad(edge_ref[...].T, ((0, 0), (0, NTYPES_N - NTYPES_E)))
    tab_ref[...] = jnp.stack([node_t, edge_t])
    pos_ref[...] = jnp.stack([posn_ref[...], pose_ref[...]])


_prep = pl.pallas_call(
    _prep_body,
    out_shape=(
        jax.ShapeDtypeStruct((SEQLEN, CHUNK, D), jnp.float32),
        jax.ShapeDtypeStruct((2, D, NTYPES_N), jnp.float32),
        jax.ShapeDtypeStruct((2, SEQLEN, D), jnp.float32),
    ),
)


def _embed_body(idx_ref, tab_ref, pos_ref, out_ref):
    idx = idx_ref[0, 0, 0, :]
    oh = (lax.iota(jnp.int32, NTYPES_N)[:, None]
          == idx[None, :]).astype(jnp.float32)
    res = jnp.dot(tab_ref[0], oh, precision=lax.Precision.HIGHEST,
                  preferred_element_type=jnp.float32)
    out_ref[0] = res + pos_ref[0, 0, 0, :][:, None]


def _embed_body2(idx_ref, tab_ref, pos_ref, base_ref, out_ref):
    _embed_body(idx_ref, tab_ref, pos_ref, out_ref)


_embed = pl.pallas_call(
    _embed_body2,
    grid=(2 * SEQLEN,),
    in_specs=[
        pl.BlockSpec((1, 1, 1, BATCH),
                     lambda i: (i // SEQLEN, i % SEQLEN, 0, 0)),
        pl.BlockSpec((1, D, NTYPES_N), lambda i: (i // SEQLEN, 0, 0)),
        pl.BlockSpec((1, 1, 1, D),
                     lambda i: (i // SEQLEN, i % SEQLEN, 0, 0)),
        pl.BlockSpec(memory_space=pltpu.MemorySpace.HBM),
    ],
    out_specs=pl.BlockSpec((1, D, BATCH), lambda i: (i, 0, 0)),
    out_shape=jax.ShapeDtypeStruct((3 * SEQLEN, D, BATCH), jnp.float32),
    input_output_aliases={3: 0},
)

_vmesh = plsc.VectorSubcoreMesh(core_axis_name="c", subcore_axis_name="s")


@functools.partial(
    pl.kernel,
    out_type=jax.ShapeDtypeStruct((3 * SEQLEN, D, BATCH), jnp.float32),
    mesh=_vmesh,
    compiler_params=(
        dataclasses.replace(
            pltpu.CompilerParams(use_tc_tiling_on_sc=False),
            needs_layout_passes=False)
        if "needs_layout_passes" in pltpu.CompilerParams.__dataclass_fields__
        else pltpu.CompilerParams(use_tc_tiling_on_sc=False)),
    scratch_types=[
        pltpu.VMEM((4, 128), jnp.int32),
        pltpu.VMEM((WB, D), jnp.float32),
        pltpu.VMEM((WB,), jnp.int32),
        pltpu.VMEM((4, 128), jnp.int32),
        pltpu.VMEM((48,), jnp.int32),
        pltpu.VMEM((WB + 16,), jnp.float32),
        pltpu.VMEM((CHUNK, D), jnp.float32),
        pltpu.VMEM((D, CHUNK), jnp.float32),
        pltpu.VMEM_SHARED((RB + 8, D), jnp.float32),
        pltpu.SemaphoreType.DMA,
    ],
)
def _sc_spmm(rows_h, cols_h, vals_h, vembed_h, posrep_h, bounds_h,
             out_h, idx2, gbuf, rowstage, rowsbuf,
             bounds_s, vals_s, tbuf_in, tbuf_out, acc, sem):
    c = lax.axis_index("c")
    t = lax.axis_index("s")
    lane = lax.iota(jnp.int32, 16)

    pltpu.sync_copy(bounds_h, bounds_s.at[pl.ds(0, 32)])

    for bi in range(NBLK // NC):
        b = bi * NC + c
        r0 = b * RB
        sb = bounds_s[pl.ds(b, 16)][0]
        eb = bounds_s[pl.ds(b + 1, 16)][0]

        @pl.loop(t, NCHUNK, step=NS)
        def _(k):
            seqq = lax.div(r0 + k * CHUNK, BATCH)
            pltpu.sync_copy(posrep_h.at[seqq], acc.at[pl.ds(k * CHUNK, CHUNK)])
        plsc.subcore_barrier()

        jlo = lax.div(sb, WB)
        jhi = lax.div(eb + (WB - 1), WB)

        @pl.loop(jlo + t, jhi, step=NS)
        def _(j):
            base = j * WB
            for q in range(WB // 128):
                pltpu.sync_copy(cols_h.at[pl.ds(base + q * 128, 128)],
                                idx2.at[q])
            pltpu.sync_copy(rows_h.at[pl.ds(base, WB)], rowstage)
            pltpu.sync_copy(vals_h.at[pl.ds(base, WB)],
                            vals_s.at[pl.ds(0, WB)])
            for q in range(WB // 128):
                for g8 in range(8):
                    g = q * 8 + g8
                    rv = rowstage[pl.ds(g * 16, 16)] - r0
                    oob = (rv < 0) | (rv >= RB)
                    rowsbuf[q, pl.ds(g8 * 16, 16)] = jnp.where(oob, RB, rv)
            cps = [
                pltpu.async_copy(vembed_h.at[idx2.at[q]],
                                 gbuf.at[pl.ds(q * 128, 128)], sem)
                for q in range(WB // 128)
            ]
            for cp in cps:
                cp.wait()

            @pl.loop(0, WB)
            def _(i):
                v = vals_s[pl.ds(i, 16)][0]
                for qq in range(D // 16):
                    slq = pl.ds(qq * 16, 16)
                    gbuf[i, slq] = gbuf[i, slq] * v

            for q in range(WB // 128):
                pltpu.sync_copy(gbuf.at[pl.ds(q * 128, 128)],
                                acc.at[rowsbuf.at[q]], add=True)

        plsc.subcore_barrier()
        @pl.loop(t, NCHUNK, step=NS)
        def _(k):
            lrow = k * CHUNK
            g = r0 + lrow
            seqq = 2 * SEQLEN + lax.div(g, BATCH)
            col0 = lax.rem(g, BATCH)
            pltpu.sync_copy(acc.at[pl.ds(lrow, CHUNK)], tbuf_in)

            @pl.loop(0, CHUNK // 16)
            def _(rg):
                rvec = rg * 16 + lane
                for e in range(D):
                    evec = jnp.broadcast_to(e, (16,)).astype(jnp.int32)
                    tbuf_out[e, pl.ds(rg * 16, 16)] = plsc.load_gather(
                        tbuf_in, [rvec, evec])

            pltpu.sync_copy(tbuf_out,
                            out_h.at[seqq].at[:, pl.ds(col0, CHUNK)])
        plsc.subcore_barrier()


def kernel(node_idx, edge_idx, val_rows, val_cols, val_vals,
           node_table, edge_table, val_embed, pos_node, pos_edge, pos_val):
    posrep, tabs, poss = _prep(node_table, edge_table,
                               pos_node, pos_edge, pos_val)
    idx2 = jnp.stack([node_idx, edge_idx]).astype(jnp.int32)
    idx2 = idx2.reshape(2, SEQLEN, 1, BATCH)
    rows = val_rows.astype(jnp.int32)
    cols = val_cols.astype(jnp.int32)
    edges = jnp.arange(0, NROWS + 1, RB, dtype=jnp.int32)
    bounds = jnp.searchsorted(rows, edges).astype(jnp.int32)
    bounds = jnp.concatenate(
        [bounds, jnp.zeros((32 - NBLK - 1,), jnp.int32)])
    out_t = _sc_spmm(rows, cols, val_vals, val_embed, posrep, bounds)
    out_t = _embed(idx2, tabs, poss.reshape(2, SEQLEN, 1, D), out_t)
    return jnp.transpose(out_t, (0, 2, 1))


# speedup vs baseline: 1.2914x; 1.2914x over previous
import dataclasses
import functools

import jax
import jax.numpy as jnp
from jax import lax
from jax.experimental import pallas as pl
from jax.experimental.pallas import tpu as pltpu
from jax.experimental.pallas import tpu_sc as plsc

SEQLEN = 50
BATCH = 4096
D = 64
NTYPES_N = 100
NTYPES_E = 50
NROWS = SEQLEN * BATCH
TOTNNZ = 409600
NBLK = 40
RB = NROWS // NBLK
CHUNK = 256
NCHUNK = RB // CHUNK
WB = 512
NC = 2
NS = 16
ROWS_PER_TILE = RB // NS


def _prep_body(node_ref, edge_ref, posn_ref, pose_ref, posv_ref,
               pr_ref, tab_ref, pos_ref):
    posv_pad = jnp.pad(posv_ref[...], ((0, 0), (0, D)))
    pr_ref[...] = jnp.broadcast_to(posv_pad[:, None, :],
                                   (SEQLEN, CHUNK, 2 * D))
    node_t = node_ref[...].T
    edge_t = jnp.pad(edge_ref[...].T, ((0, 0), (0, NTYPES_N - NTYPES_E)))
    tab_ref[...] = jnp.stack([node_t, edge_t])
    pos_ref[...] = jnp.stack([posn_ref[...], pose_ref[...]])


_prep = pl.pallas_call(
    _prep_body,
    out_shape=(
        jax.ShapeDtypeStruct((SEQLEN, CHUNK, 2 * D), jnp.float32),
        jax.ShapeDtypeStruct((2, D, NTYPES_N), jnp.float32),
        jax.ShapeDtypeStruct((2, SEQLEN, D), jnp.float32),
    ),
)


def _vpad_body(v_ref, out_ref):
    out_ref[...] = jnp.pad(v_ref[...], ((0, 0), (0, D)))


_vpad = pl.pallas_call(
    _vpad_body,
    grid=(50,),
    in_specs=[pl.BlockSpec((2000, D), lambda i: (i, 0))],
    out_specs=pl.BlockSpec((2000, 2 * D), lambda i: (i, 0)),
    out_shape=jax.ShapeDtypeStruct((100000, 2 * D), jnp.float32),
)


def _embed_body(idx_ref, tab_ref, pos_ref, out_ref):
    idx = idx_ref[0, 0, 0, :]
    oh = (lax.iota(jnp.int32, NTYPES_N)[:, None]
          == idx[None, :]).astype(jnp.float32)
    res = jnp.dot(tab_ref[0], oh, precision=lax.Precision.HIGHEST,
                  preferred_element_type=jnp.float32)
    out_ref[0] = res + pos_ref[0, 0, 0, :][:, None]


_embed = pl.pallas_call(
    _embed_body,
    grid=(2 * SEQLEN,),
    in_specs=[
        pl.BlockSpec((1, 1, 1, BATCH),
                     lambda i: (i // SEQLEN, i % SEQLEN, 0, 0)),
        pl.BlockSpec((1, D, NTYPES_N), lambda i: (i // SEQLEN, 0, 0)),
        pl.BlockSpec((1, 1, 1, D),
                     lambda i: (i // SEQLEN, i % SEQLEN, 0, 0)),
    ],
    out_specs=pl.BlockSpec((1, D, BATCH), lambda i: (i, 0, 0)),
    out_shape=jax.ShapeDtypeStruct((3 * SEQLEN, D, BATCH), jnp.float32),
)


def _valpack_body(v_ref, base_ref, out_ref):
    out_ref[0] = v_ref[0][:, :D].T


_valpack = pl.pallas_call(
    _valpack_body,
    grid=(SEQLEN,),
    in_specs=[
        pl.BlockSpec((1, BATCH, 2 * D), lambda i: (i, 0, 0)),
        pl.BlockSpec(memory_space=pltpu.MemorySpace.HBM),
    ],
    out_specs=pl.BlockSpec((1, D, BATCH), lambda i: (2 * SEQLEN + i, 0, 0)),
    out_shape=jax.ShapeDtypeStruct((3 * SEQLEN, D, BATCH), jnp.float32),
    input_output_aliases={1: 0},
)

_vmesh = plsc.VectorSubcoreMesh(core_axis_name="c", subcore_axis_name="s")


@functools.partial(
    pl.kernel,
    out_type=jax.ShapeDtypeStruct((SEQLEN, BATCH, 2 * D), jnp.float32),
    mesh=_vmesh,
    scratch_types=[
        pltpu.VMEM((4, 128), jnp.int32),
        pltpu.VMEM((WB, 2 * D), jnp.float32),
        pltpu.VMEM((WB,), jnp.int32),
        pltpu.VMEM((4, 128), jnp.int32),
        pltpu.VMEM((64,), jnp.int32),
        pltpu.VMEM((WB + 16,), jnp.float32),
        pltpu.VMEM_SHARED((RB + 8, 2 * D), jnp.float32),
        pltpu.SemaphoreType.DMA,
    ],
)
def _sc_spmm(rows_h, cols_h, vals_h, vembed_h, posrep_h, bounds_h,
             out_h, idx2, gbuf, rowstage, rowsbuf,
             bounds_s, vals_s, acc, sem):
    c = lax.axis_index("c")
    t = lax.axis_index("s")

    pltpu.sync_copy(bounds_h, bounds_s.at[pl.ds(0, 48)])

    for bi in range(NBLK // NC):
        b = bi * NC + c
        r0 = b * RB
        sb = bounds_s[pl.ds(b, 16)][0]
        eb = bounds_s[pl.ds(b + 1, 16)][0]

        @pl.loop(t, NCHUNK, step=NS)
        def _(k):
            seqq = lax.div(r0 + k * CHUNK, BATCH)
            pltpu.sync_copy(posrep_h.at[seqq], acc.at[pl.ds(k * CHUNK, CHUNK)])
        plsc.subcore_barrier()

        jlo = lax.div(sb, WB)
        jhi = lax.div(eb + (WB - 1), WB)

        @pl.loop(jlo + t, jhi, step=NS)
        def _(j):
            base = j * WB
            for q in range(WB // 128):
                pltpu.sync_copy(cols_h.at[pl.ds(base + q * 128, 128)],
                                idx2.at[q])
            pltpu.sync_copy(rows_h.at[pl.ds(base, WB)], rowstage)
            pltpu.sync_copy(vals_h.at[pl.ds(base, WB)],
                            vals_s.at[pl.ds(0, WB)])
            for q in range(WB // 128):
                for g8 in range(8):
                    g = q * 8 + g8
                    rv = rowstage[pl.ds(g * 16, 16)] - r0
                    oob = (rv < 0) | (rv >= RB)
                    rowsbuf[q, pl.ds(g8 * 16, 16)] = jnp.where(oob, RB, rv)
            cps = [
                pltpu.async_copy(vembed_h.at[idx2.at[q]],
                                 gbuf.at[pl.ds(q * 128, 128)], sem)
                for q in range(WB // 128)
            ]
            for cp in cps:
                cp.wait()

            @pl.loop(0, WB)
            def _(i):
                v = vals_s[pl.ds(i, 16)][0]
                for qq in range(D // 16):
                    slq = pl.ds(qq * 16, 16)
                    gbuf[i, slq] = gbuf[i, slq] * v

            for q in range(WB // 128):
                pltpu.sync_copy(gbuf.at[pl.ds(q * 128, 128)],
                                acc.at[rowsbuf.at[q]], add=True)

        plsc.subcore_barrier()
        @pl.loop(t, NCHUNK, step=NS)
        def _(k):
            lrow = k * CHUNK
            g = r0 + lrow
            pltpu.sync_copy(
                acc.at[pl.ds(lrow, CHUNK)],
                out_h.at[lax.div(g, BATCH)].at[pl.ds(lax.rem(g, BATCH),
                                                     CHUNK)])
        plsc.subcore_barrier()


def kernel(node_idx, edge_idx, val_rows, val_cols, val_vals,
           node_table, edge_table, val_embed, pos_node, pos_edge, pos_val):
    posrep, tabs, poss = _prep(node_table, edge_table,
                               pos_node, pos_edge, pos_val)
    idx2 = jnp.stack([node_idx, edge_idx]).astype(jnp.int32)
    idx2 = idx2.reshape(2, SEQLEN, 1, BATCH)
    rows = val_rows.astype(jnp.int32)
    cols = val_cols.astype(jnp.int32)
    edges = jnp.arange(0, NROWS + 1, RB, dtype=jnp.int32)
    bounds = jnp.searchsorted(rows, edges).astype(jnp.int32)
    bounds = jnp.concatenate(
        [bounds, jnp.zeros((48 - NBLK - 1,), jnp.int32)])
    vpad = _vpad(val_embed)
    emb = _embed(idx2, tabs, poss.reshape(2, SEQLEN, 1, D))
    val128 = _sc_spmm(rows, cols, val_vals, vpad, posrep, bounds)
    out_t = _valpack(val128, emb)
    return jnp.transpose(out_t, (0, 2, 1))


# speedup vs baseline: 1.6296x; 1.2619x over previous
import dataclasses
import functools

import jax
import jax.numpy as jnp
from jax import lax
from jax.experimental import pallas as pl
from jax.experimental.pallas import tpu as pltpu
from jax.experimental.pallas import tpu_sc as plsc

SEQLEN = 50
BATCH = 4096
D = 64
NTYPES_N = 100
NTYPES_E = 50
NROWS = SEQLEN * BATCH
TOTNNZ = 409600
NBLK = 40
RB = NROWS // NBLK
CHUNK = 256
NCHUNK = RB // CHUNK
WB = 512
NC = 2
NS = 16
ROWS_PER_TILE = RB // NS


def _prep_body(node_ref, edge_ref, posn_ref, pose_ref, posv_ref,
               pr_ref, tab_ref, pos_ref):
    posv_pad = jnp.pad(posv_ref[...], ((0, 0), (0, D)))
    pr_ref[...] = jnp.broadcast_to(posv_pad[:, None, :],
                                   (SEQLEN, CHUNK, 2 * D))
    node_t = node_ref[...].T
    edge_t = jnp.pad(edge_ref[...].T, ((0, 0), (0, NTYPES_N - NTYPES_E)))
    tab_ref[...] = jnp.stack([node_t, edge_t])
    pos_ref[...] = jnp.stack([posn_ref[...], pose_ref[...]])


_prep = pl.pallas_call(
    _prep_body,
    out_shape=(
        jax.ShapeDtypeStruct((SEQLEN, CHUNK, 2 * D), jnp.float32),
        jax.ShapeDtypeStruct((2, D, NTYPES_N), jnp.float32),
        jax.ShapeDtypeStruct((2, SEQLEN, D), jnp.float32),
    ),
)


def _vpad_body(v_ref, out_ref):
    out_ref[...] = jnp.pad(v_ref[...], ((0, 0), (0, D)))


_vpad = pl.pallas_call(
    _vpad_body,
    grid=(50,),
    in_specs=[pl.BlockSpec((2000, D), lambda i: (i, 0))],
    out_specs=pl.BlockSpec((2000, 2 * D), lambda i: (i, 0)),
    out_shape=jax.ShapeDtypeStruct((100000, 2 * D), jnp.float32),
)


def _embed_body(idx_ref, tab_ref, pos_ref, out_ref):
    idx = idx_ref[0, 0, 0, :]
    oh = (lax.iota(jnp.int32, NTYPES_N)[:, None]
          == idx[None, :]).astype(jnp.float32)
    res = jnp.dot(tab_ref[0], oh, precision=lax.Precision.HIGHEST,
                  preferred_element_type=jnp.float32)
    out_ref[0] = res + pos_ref[0, 0, 0, :][:, None]


_embed = pl.pallas_call(
    _embed_body,
    grid=(2 * SEQLEN,),
    in_specs=[
        pl.BlockSpec((1, 1, 1, BATCH),
                     lambda i: (i // SEQLEN, i % SEQLEN, 0, 0)),
        pl.BlockSpec((1, D, NTYPES_N), lambda i: (i // SEQLEN, 0, 0)),
        pl.BlockSpec((1, 1, 1, D),
                     lambda i: (i // SEQLEN, i % SEQLEN, 0, 0)),
    ],
    out_specs=pl.BlockSpec((1, D, BATCH), lambda i: (i, 0, 0)),
    out_shape=jax.ShapeDtypeStruct((3 * SEQLEN, D, BATCH), jnp.float32),
)


def _valpack_body(v_ref, base_ref, out_ref):
    out_ref[0] = v_ref[0][:, :D].T


_valpack = pl.pallas_call(
    _valpack_body,
    grid=(SEQLEN,),
    in_specs=[
        pl.BlockSpec((1, BATCH, 2 * D), lambda i: (i, 0, 0)),
        pl.BlockSpec(memory_space=pltpu.MemorySpace.HBM),
    ],
    out_specs=pl.BlockSpec((1, D, BATCH), lambda i: (2 * SEQLEN + i, 0, 0)),
    out_shape=jax.ShapeDtypeStruct((3 * SEQLEN, D, BATCH), jnp.float32),
    input_output_aliases={1: 0},
)

_vmesh = plsc.VectorSubcoreMesh(core_axis_name="c", subcore_axis_name="s")


@functools.partial(
    pl.kernel,
    out_type=jax.ShapeDtypeStruct((SEQLEN, BATCH, 2 * D), jnp.float32),
    mesh=_vmesh,
    scratch_types=[
        pltpu.VMEM((4, 128), jnp.int32),
        pltpu.VMEM((WB, 2 * D), jnp.float32),
        pltpu.VMEM((WB,), jnp.int32),
        pltpu.VMEM((4, 128), jnp.int32),
        pltpu.VMEM((64,), jnp.int32),
        pltpu.VMEM((WB + 16,), jnp.float32),
        pltpu.VMEM_SHARED((RB + 8, 2 * D), jnp.float32),
        pltpu.SemaphoreType.DMA,
    ],
)
def _sc_spmm(rows_h, cols_h, vals_h, vembed_h, posrep_h, bounds_h,
             out_h, idx2, gbuf, rowstage, rowsbuf,
             bounds_s, vals_s, acc, sem):
    c = lax.axis_index("c")
    t = lax.axis_index("s")

    pltpu.sync_copy(bounds_h, bounds_s.at[pl.ds(0, 48)])

    for bi in range(NBLK // NC):
        b = bi * NC + c
        r0 = b * RB
        sb = bounds_s[pl.ds(b, 16)][0]
        eb = bounds_s[pl.ds(b + 1, 16)][0]

        @pl.loop(t, NCHUNK, step=NS)
        def _(k):
            seqq = lax.div(r0 + k * CHUNK, BATCH)
            pltpu.sync_copy(posrep_h.at[seqq], acc.at[pl.ds(k * CHUNK, CHUNK)])
        plsc.subcore_barrier()

        jlo = lax.div(sb, WB)
        jhi = lax.div(eb + (WB - 1), WB)

        @pl.loop(jlo + t, jhi, step=NS)
        def _(j):
            base = j * WB
            for q in range(WB // 128):
                pltpu.sync_copy(cols_h.at[pl.ds(base + q * 128, 128)],
                                idx2.at[q])
            pltpu.sync_copy(rows_h.at[pl.ds(base, WB)], rowstage)
            pltpu.sync_copy(vals_h.at[pl.ds(base, WB)],
                            vals_s.at[pl.ds(0, WB)])
            for q in range(WB // 128):
                for g8 in range(8):
                    g = q * 8 + g8
                    rv = rowstage[pl.ds(g * 16, 16)] - r0
                    oob = (rv < 0) | (rv >= RB)
                    rowsbuf[q, pl.ds(g8 * 16, 16)] = jnp.where(oob, RB, rv)
            cps = [
                pltpu.async_copy(vembed_h.at[idx2.at[q]],
                                 gbuf.at[pl.ds(q * 128, 128)], sem)
                for q in range(WB // 128)
            ]
            for cp in cps:
                cp.wait()

            @functools.partial(plsc.parallel_loop, 0, WB, unroll=4)
            def _(i):
                v = vals_s[pl.ds(i, 16)][0]
                for qq in range(D // 16):
                    slq = pl.ds(qq * 16, 16)
                    gbuf[i, slq] = gbuf[i, slq] * v

            for q in range(WB // 128):
                pltpu.sync_copy(gbuf.at[pl.ds(q * 128, 128)],
                                acc.at[rowsbuf.at[q]], add=True)

        plsc.subcore_barrier()
        @pl.loop(t, NCHUNK, step=NS)
        def _(k):
            lrow = k * CHUNK
            g = r0 + lrow
            pltpu.sync_copy(
                acc.at[pl.ds(lrow, CHUNK)],
                out_h.at[lax.div(g, BATCH)].at[pl.ds(lax.rem(g, BATCH),
                                                     CHUNK)])
        plsc.subcore_barrier()


def kernel(node_idx, edge_idx, val_rows, val_cols, val_vals,
           node_table, edge_table, val_embed, pos_node, pos_edge, pos_val):
    posrep, tabs, poss = _prep(node_table, edge_table,
                               pos_node, pos_edge, pos_val)
    idx2 = jnp.stack([node_idx, edge_idx]).astype(jnp.int32)
    idx2 = idx2.reshape(2, SEQLEN, 1, BATCH)
    rows = val_rows.astype(jnp.int32)
    cols = val_cols.astype(jnp.int32)
    edges = jnp.arange(0, NROWS + 1, RB, dtype=jnp.int32)
    bounds = jnp.sum(rows[:, None] < edges[None, :], axis=0,
                     dtype=jnp.int32)
    bounds = jnp.concatenate(
        [bounds, jnp.zeros((48 - NBLK - 1,), jnp.int32)])
    vpad = _vpad(val_embed)
    emb = _embed(idx2, tabs, poss.reshape(2, SEQLEN, 1, D))
    val128 = _sc_spmm(rows, cols, val_vals, vpad, posrep, bounds)
    out_t = _valpack(val128, emb)
    return jnp.transpose(out_t, (0, 2, 1))
